# Initial kernel scaffold; baseline (speedup 1.0000x reference)
import jax
import jax.numpy as jnp
from jax.experimental import pallas as pl


def kernel(x, edge_index, Wp, bp, ln_g0, ln_b0, Wl0, Wr0, att0, bias0, ln_g1, ln_b1, Wl1, Wr1, att1, bias1, Wc, bc):
    raise NotImplementedError("write your pallas kernel here")


# speedup vs baseline: 3.2091x; 3.2091x over previous
import functools

import jax
import jax.numpy as jnp
from jax import lax
from jax.experimental import pallas as pl
from jax.experimental.pallas import tpu as pltpu
from jax.experimental.pallas import tpu_sc as plsc

N = 10000
D_IN = 128
HID = 128
FEAT = 128
N_CLUSTERS = 64
E_RAW = 320000
E_TOT = E_RAW + N

NP = 10240
DW = 144
DUMMY = N + 16

NW = 32
C = 80
EP = ((E_TOT + NW * C - 1) // (NW * C)) * (NW * C)
E_PER_W = EP // NW
N_CHUNK = E_PER_W // C
RPT = NP // 16

BM = 1024


def _layer_norm(h, g, b):
    mu = jnp.mean(h, axis=-1, keepdims=True)
    var = jnp.mean((h - mu) ** 2, axis=-1, keepdims=True)
    return (h - mu) / jnp.sqrt(var + 1e-5) * g + b


def _pack144(xl):
    bm = xl.shape[0]
    tag = jnp.where(lax.broadcasted_iota(jnp.int32, (bm, 16), 1) == 0, 1.0, 0.0)
    return jnp.concatenate([xl, tag.astype(jnp.float32)], axis=1)


def _ka_body(x_ref, wp_ref, bp_ref, g_ref, b_ref, wl_ref, wr_ref,
             xlp_ref, xr_ref):
    h = jnp.dot(x_ref[...], wp_ref[...], preferred_element_type=jnp.float32)
    h = h + bp_ref[...]
    hn = _layer_norm(h, g_ref[...], b_ref[...])
    xl = jnp.dot(hn, wl_ref[...], preferred_element_type=jnp.float32)
    xr = jnp.dot(hn, wr_ref[...], preferred_element_type=jnp.float32)
    xlp_ref[...] = _pack144(xl)
    xr_ref[...] = xr


def _tc_proj_first(x, Wp, bp, g, b, Wl, Wr):
    return pl.pallas_call(
        _ka_body,
        grid=(NP // BM,),
        in_specs=[
            pl.BlockSpec((BM, D_IN), lambda i: (i, 0)),
            pl.BlockSpec((D_IN, FEAT), lambda i: (0, 0)),
            pl.BlockSpec((1, FEAT), lambda i: (0, 0)),
            pl.BlockSpec((1, FEAT), lambda i: (0, 0)),
            pl.BlockSpec((1, FEAT), lambda i: (0, 0)),
            pl.BlockSpec((FEAT, FEAT), lambda i: (0, 0)),
            pl.BlockSpec((FEAT, FEAT), lambda i: (0, 0)),
        ],
        out_specs=[
            pl.BlockSpec((BM, DW), lambda i: (i, 0)),
            pl.BlockSpec((BM, FEAT), lambda i: (i, 0)),
        ],
        out_shape=[
            jax.ShapeDtypeStruct((NP, DW), jnp.float32),
            jax.ShapeDtypeStruct((NP, FEAT), jnp.float32),
        ],
    )(x, Wp, bp, g, b, Wl, Wr)


def _combine(acc_ref, bias_ref):
    s = acc_ref[0] + acc_ref[1]
    num = s[:, :FEAT]
    den = s[:, FEAT:FEAT + 1]
    return jnp.maximum(num / (den + 1e-16) + bias_ref[...], 0.0)


def _kc_body(acc_ref, bias_ref, g_ref, b_ref, wl_ref, wr_ref,
             xlp_ref, xr_ref):
    h = _combine(acc_ref, bias_ref)
    hn = _layer_norm(h, g_ref[...], b_ref[...])
    xl = jnp.dot(hn, wl_ref[...], preferred_element_type=jnp.float32)
    xr = jnp.dot(hn, wr_ref[...], preferred_element_type=jnp.float32)
    xlp_ref[...] = _pack144(xl)
    xr_ref[...] = xr


def _tc_combine_mid(acc, bias, g, b, Wl, Wr):
    return pl.pallas_call(
        _kc_body,
        grid=(NP // BM,),
        in_specs=[
            pl.BlockSpec((2, BM, DW), lambda i: (0, i, 0)),
            pl.BlockSpec((1, FEAT), lambda i: (0, 0)),
            pl.BlockSpec((1, FEAT), lambda i: (0, 0)),
            pl.BlockSpec((1, FEAT), lambda i: (0, 0)),
            pl.BlockSpec((FEAT, FEAT), lambda i: (0, 0)),
            pl.BlockSpec((FEAT, FEAT), lambda i: (0, 0)),
        ],
        out_specs=[
            pl.BlockSpec((BM, DW), lambda i: (i, 0)),
            pl.BlockSpec((BM, FEAT), lambda i: (i, 0)),
        ],
        out_shape=[
            jax.ShapeDtypeStruct((NP, DW), jnp.float32),
            jax.ShapeDtypeStruct((NP, FEAT), jnp.float32),
        ],
    )(acc, bias, g, b, Wl, Wr)


def _kd_body(acc_ref, bias_ref, wc_ref, bc_ref, h_ref, cls_ref):
    h = _combine(acc_ref, bias_ref)
    h_ref[...] = h
    cls_ref[...] = jnp.dot(h, wc_ref[...], preferred_element_type=jnp.float32) + bc_ref[...]


def _tc_combine_last(acc, bias, Wc, bc):
    return pl.pallas_call(
        _kd_body,
        grid=(NP // BM,),
        in_specs=[
            pl.BlockSpec((2, BM, DW), lambda i: (0, i, 0)),
            pl.BlockSpec((1, FEAT), lambda i: (0, 0)),
            pl.BlockSpec((FEAT, N_CLUSTERS), lambda i: (0, 0)),
            pl.BlockSpec((1, N_CLUSTERS), lambda i: (0, 0)),
        ],
        out_specs=[
            pl.BlockSpec((BM, FEAT), lambda i: (i, 0)),
            pl.BlockSpec((BM, N_CLUSTERS), lambda i: (i, 0)),
        ],
        out_shape=[
            jax.ShapeDtypeStruct((NP, FEAT), jnp.float32),
            jax.ShapeDtypeStruct((NP, N_CLUSTERS), jnp.float32),
        ],
    )(acc, bias, Wc, bc)


_MESH = plsc.VectorSubcoreMesh(core_axis_name="c", subcore_axis_name="s")


@functools.partial(
    pl.kernel,
    out_type=jax.ShapeDtypeStruct((2, NP, DW), jnp.float32),
    mesh=_MESH,
    compiler_params=pltpu.CompilerParams(
        use_tc_tiling_on_sc=False, needs_layout_passes=False),
    scratch_types=[
        pltpu.VMEM((C,), jnp.int32),
        pltpu.VMEM((C,), jnp.int32),
        pltpu.VMEM((C, DW), jnp.float32),
        pltpu.VMEM((C, FEAT), jnp.float32),
        pltpu.VMEM((FEAT,), jnp.float32),
        pltpu.VMEM_SHARED((NP, DW), jnp.float32),
    ],
)
def _sc_edge_kernel(xlp_hbm, xr_hbm, att_hbm, src_hbm, dst_hbm, zer_hbm,
                    out_hbm, idx_s, idx_d, rows_s, rows_d, att_v, acc_sh):
    cid = lax.axis_index("c")
    sid = lax.axis_index("s")
    w = sid * 2 + cid

    pltpu.sync_copy(zer_hbm.at[pl.ds(sid * RPT, RPT)],
                    acc_sh.at[pl.ds(sid * RPT, RPT)])
    pltpu.sync_copy(att_hbm, att_v)
    plsc.subcore_barrier()

    lane = lax.iota(jnp.int32, 16)

    def chunk_body(ci, carry):
        base = w * E_PER_W + ci * C
        pltpu.sync_copy(src_hbm.at[pl.ds(base, C)], idx_s)
        pltpu.sync_copy(dst_hbm.at[pl.ds(base, C)], idx_d)
        pltpu.sync_copy(xlp_hbm.at[idx_s], rows_s)
        pltpu.sync_copy(xr_hbm.at[idx_d], rows_d)

        def group_body(g, carry2):
            rowid = g * 16 + lane

            def dot_body(f, acc):
                col = jnp.full((16,), f, jnp.int32)
                a_s = plsc.load_gather(rows_s, [rowid, col])
                a_d = plsc.load_gather(rows_d, [rowid, col])
                af = plsc.load_gather(att_v, [col])
                z = a_s + a_d
                m = jnp.maximum(z, 0.2 * z)
                return acc + m * af

            e = lax.fori_loop(0, FEAT, dot_body, jnp.zeros((16,), jnp.float32))
            ee = jnp.exp(e)

            def scale_body(f, carry3):
                col = jnp.full((16,), f, jnp.int32)
                v = plsc.load_gather(rows_s, [rowid, col])
                plsc.store_scatter(rows_s, [rowid, col], v * ee)
                return carry3

            lax.fori_loop(0, DW, scale_body, 0)
            return carry2

        lax.fori_loop(0, C // 16, group_body, 0)
        pltpu.sync_copy(rows_s, acc_sh.at[idx_d], add=True)
        return carry

    lax.fori_loop(0, N_CHUNK, chunk_body, 0)
    plsc.subcore_barrier()
    pltpu.sync_copy(acc_sh.at[pl.ds(sid * RPT, RPT)],
                    out_hbm.at[cid, pl.ds(sid * RPT, RPT)])


def kernel(x, edge_index, Wp, bp, ln_g0, ln_b0, Wl0, Wr0, att0, bias0,
           ln_g1, ln_b1, Wl1, Wr1, att1, bias1, Wc, bc):
    loop = jnp.arange(N, dtype=edge_index.dtype)
    src = jnp.concatenate(
        [edge_index[0], loop,
         jnp.full((EP - E_TOT,), DUMMY, dtype=edge_index.dtype)])
    dst = jnp.concatenate(
        [edge_index[1], loop,
         jnp.full((EP - E_TOT,), DUMMY, dtype=edge_index.dtype)])

    x_pad = jnp.pad(x, ((0, NP - N), (0, 0)))
    zer = jnp.zeros((NP, DW), jnp.float32)

    r2 = lambda v: v.reshape(1, -1)

    xlp, xr = _tc_proj_first(x_pad, Wp, r2(bp), r2(ln_g0), r2(ln_b0), Wl0, Wr0)
    acc0 = _sc_edge_kernel(xlp, xr, att0.reshape(-1), src, dst, zer)
    xlp1, xr1 = _tc_combine_mid(acc0, r2(bias0), r2(ln_g1), r2(ln_b1), Wl1, Wr1)
    acc1 = _sc_edge_kernel(xlp1, xr1, att1.reshape(-1), src, dst, zer)
    h2, cls = _tc_combine_last(acc1, r2(bias1), Wc, r2(bc))

    return (cls[:N], h2[:N])
